# baseline (speedup 1.0000x reference)
import jax
import jax.numpy as jnp
from jax.experimental import pallas as pl


def _blk(n, target):
    b = min(n, target)
    b -= b % 8
    while b > 8 and n % b:
        b -= 8
    return max(b, 8)


def _mm(a, w):
    return jnp.dot(a, w, preferred_element_type=jnp.float32)


def _stats(z, s_ref, ss_ref):
    @pl.when(pl.program_id(0) == 0)
    def _init():
        s_ref[...] = jnp.zeros_like(s_ref)
        ss_ref[...] = jnp.zeros_like(ss_ref)

    s_ref[...] += jnp.sum(z, axis=0, keepdims=True)
    ss_ref[...] += jnp.sum(z * z, axis=0, keepdims=True)


def _edge_lin_body(hi_ref, hj_ref, ec_ref, wa_ref, wb_ref, wc_ref, b_ref,
                   z_ref, s_ref, ss_ref):
    z = (_mm(hi_ref[...], wa_ref[...]) + _mm(hj_ref[...], wb_ref[...])
         + _mm(ec_ref[...], wc_ref[...]) + b_ref[...])
    z_ref[...] = z
    _stats(z, s_ref, ss_ref)


def _node_lin_body(h_ref, g_ref, wa_ref, wb_ref, b_ref, z_ref, s_ref, ss_ref):
    z = (_mm(h_ref[...], wa_ref[...]) + _mm(g_ref[...], wb_ref[...])
         + b_ref[...])
    z_ref[...] = z
    _stats(z, s_ref, ss_ref)


def _act_lin_body(z1_ref, a_ref, c_ref, w_ref, b_ref, z2_ref, s_ref, ss_ref):
    m = jnp.maximum(z1_ref[...] * a_ref[...] + c_ref[...], 0.0)
    z = _mm(m, w_ref[...]) + b_ref[...]
    z2_ref[...] = z
    _stats(z, s_ref, ss_ref)


def _edge_out_body(z2_ref, a_ref, c_ref, e_ref, m_ref, eo_ref):
    m = jnp.maximum(z2_ref[...] * a_ref[...] + c_ref[...], 0.0)
    m_ref[...] = m
    eo_ref[...] = e_ref[...] + m


def _edge_out0_body(z2_ref, a_ref, c_ref, ea_ref, we_ref, be_ref, m_ref,
                    eo_ref):
    m = jnp.maximum(z2_ref[...] * a_ref[...] + c_ref[...], 0.0)
    m_ref[...] = m
    eo_ref[...] = ea_ref[...] * we_ref[...] + be_ref[...] + m


def _node_out_body(z2_ref, a_ref, c_ref, h_ref, ho_ref):
    ho_ref[...] = h_ref[...] + jnp.maximum(
        z2_ref[...] * a_ref[...] + c_ref[...], 0.0)


def _final_body(z2_ref, a_ref, c_ref, e_ref, wp_ref, bp_ref, o_ref):
    m = jnp.maximum(z2_ref[...] * a_ref[...] + c_ref[...], 0.0)
    o_ref[...] = _mm(e_ref[...] + m, wp_ref[...]) + bp_ref[...]


def _proj_body(x_ref, w_ref, b_ref, h_ref):
    h_ref[...] = _mm(x_ref[...], w_ref[...]) + b_ref[...]


def _row_spec(blk, d):
    return pl.BlockSpec((blk, d), lambda i: (i, 0))


def _full_spec(shape):
    return pl.BlockSpec(shape, lambda i: (0, 0))


def _proj(x, w, b, blk):
    n, k = x.shape
    d = w.shape[1]
    return pl.pallas_call(
        _proj_body,
        grid=(n // blk,),
        in_specs=[_row_spec(blk, k), _full_spec((k, d)), _full_spec((1, d))],
        out_specs=_row_spec(blk, d),
        out_shape=jax.ShapeDtypeStruct((n, d), jnp.float32),
    )(x, w, b)


def _lin_stats_call(body, row_ins, full_ins, blk, d):
    n = row_ins[0].shape[0]
    in_specs = ([_row_spec(blk, a.shape[1]) for a in row_ins]
                + [_full_spec(a.shape) for a in full_ins])
    return pl.pallas_call(
        body,
        grid=(n // blk,),
        in_specs=in_specs,
        out_specs=[_row_spec(blk, d), _full_spec((1, d)), _full_spec((1, d))],
        out_shape=[
            jax.ShapeDtypeStruct((n, d), jnp.float32),
            jax.ShapeDtypeStruct((1, d), jnp.float32),
            jax.ShapeDtypeStruct((1, d), jnp.float32),
        ],
    )(*row_ins, *full_ins)


def _bn_affine(s, ss, cnt, g, bb):
    mu = s / cnt
    var = ss / cnt - mu * mu
    a = g[None] * jax.lax.rsqrt(var + 1e-5)
    c = bb[None] - mu * a
    return a, c


def kernel(x, edge_attr, edge_index, w_in, b_in, w_e, b_e, msg_w1, msg_b1,
           msg_g1, msg_bb1, msg_w2, msg_b2, msg_g2, msg_bb2, upd_w1, upd_b1,
           upd_g1, upd_bb1, upd_w2, upd_b2, upd_g2, upd_bb2, w_pred, b_pred):
    n = x.shape[0]
    e_cnt = edge_attr.shape[0]
    d = w_in.shape[1]
    num_layers = msg_w1.shape[0]
    src = edge_index[0]
    dst = edge_index[1]
    bn = _blk(n, 10000)
    be = _blk(e_cnt, 8000)

    h = _proj(x, w_in, b_in[None], bn)
    ea = edge_attr[:, None]
    e = None
    for l in range(num_layers):
        hi = jnp.take(h, dst, axis=0)
        hj = jnp.take(h, src, axis=0)
        wa, wb, wc = msg_w1[l][:d], msg_w1[l][d:2 * d], msg_w1[l][2 * d:]
        if l == 0:
            wc_eff = w_e @ wc
            b_eff = msg_b1[l][None] + b_e[None] @ wc
            ec = ea
        else:
            wc_eff = wc
            b_eff = msg_b1[l][None]
            ec = e
        z1, s1, ss1 = _lin_stats_call(
            _edge_lin_body, [hi, hj, ec], [wa, wb, wc_eff, b_eff], be, d)
        a1, c1 = _bn_affine(s1, ss1, e_cnt, msg_g1[l], msg_bb1[l])
        z2, s2, ss2 = _lin_stats_call(
            _act_lin_body, [z1], [a1, c1, msg_w2[l], msg_b2[l][None]], be, d)
        a2, c2 = _bn_affine(s2, ss2, e_cnt, msg_g2[l], msg_bb2[l])

        if l == num_layers - 1:
            return pl.pallas_call(
                _final_body,
                grid=(e_cnt // be,),
                in_specs=[_row_spec(be, d), _full_spec((1, d)),
                          _full_spec((1, d)), _row_spec(be, d),
                          _full_spec((d, 1)), _full_spec((1, 1))],
                out_specs=_row_spec(be, 1),
                out_shape=jax.ShapeDtypeStruct((e_cnt, 1), jnp.float32),
            )(z2, a2, c2, e, w_pred, b_pred[None])

        if l == 0:
            m, e = pl.pallas_call(
                _edge_out0_body,
                grid=(e_cnt // be,),
                in_specs=[_row_spec(be, d), _full_spec((1, d)),
                          _full_spec((1, d)), _row_spec(be, 1),
                          _full_spec((1, d)), _full_spec((1, d))],
                out_specs=[_row_spec(be, d), _row_spec(be, d)],
                out_shape=[jax.ShapeDtypeStruct((e_cnt, d), jnp.float32),
                           jax.ShapeDtypeStruct((e_cnt, d), jnp.float32)],
            )(z2, a2, c2, ea, w_e, b_e[None])
        else:
            m, e = pl.pallas_call(
                _edge_out_body,
                grid=(e_cnt // be,),
                in_specs=[_row_spec(be, d), _full_spec((1, d)),
                          _full_spec((1, d)), _row_spec(be, d)],
                out_specs=[_row_spec(be, d), _row_spec(be, d)],
                out_shape=[jax.ShapeDtypeStruct((e_cnt, d), jnp.float32),
                           jax.ShapeDtypeStruct((e_cnt, d), jnp.float32)],
            )(z2, a2, c2, e)

        agg = jax.ops.segment_sum(m, dst, num_segments=n)
        z3, s3, ss3 = _lin_stats_call(
            _node_lin_body, [h, agg],
            [upd_w1[l][:d], upd_w1[l][d:], upd_b1[l][None]], bn, d)
        a3, c3 = _bn_affine(s3, ss3, n, upd_g1[l], upd_bb1[l])
        z4, s4, ss4 = _lin_stats_call(
            _act_lin_body, [z3], [a3, c3, upd_w2[l], upd_b2[l][None]], bn, d)
        a4, c4 = _bn_affine(s4, ss4, n, upd_g2[l], upd_bb2[l])
        h = pl.pallas_call(
            _node_out_body,
            grid=(n // bn,),
            in_specs=[_row_spec(bn, d), _full_spec((1, d)),
                      _full_spec((1, d)), _row_spec(bn, d)],
            out_specs=_row_spec(bn, d),
            out_shape=jax.ShapeDtypeStruct((n, d), jnp.float32),
        )(z4, a4, c4, h)
    return None
